# baseline (speedup 1.0000x reference)
import functools

import jax
import jax.numpy as jnp
from jax import lax
from jax.experimental import pallas as pl
from jax.experimental.pallas import tpu as pltpu
from jax.experimental.pallas import tpu_sc as plsc

_SCALE = 8.0
_NW = 32
_L = 128


def _iota16():
    return lax.iota(jnp.int32, 16)


def _transpose_table(tT, tailP):
    D, V = tT.shape
    n_full = V // _L
    per_w = n_full // _NW
    n_tail = n_full - per_w * _NW
    rem = V - n_full * _L

    mesh = plsc.VectorSubcoreMesh(core_axis_name="c", subcore_axis_name="s")

    @functools.partial(
        pl.kernel,
        out_type=jax.ShapeDtypeStruct((V // 2, _L), jnp.float32),
        mesh=mesh,
        scratch_types=(
            [pltpu.VMEM((D, _L), jnp.float32) for _ in range(2)]
            + [pltpu.VMEM((_L // 2, _L), jnp.float32) for _ in range(2)]
            + [pltpu.SemaphoreType.DMA for _ in range(4)]
        ),
        compiler_params=pltpu.CompilerParams(use_tc_tiling_on_sc=True, needs_layout_passes=False),
    )
    def k(tT_hbm, tailP_hbm, tP_hbm, in0, in1, ot0, ot1, is0, is1, ws0, ws1):
        ins = (in0, in1)
        outs = (ot0, ot1)
        isem = (is0, is1)
        wsem = (ws0, ws1)
        wid = lax.axis_index("s") * 2 + lax.axis_index("c")

        ridx = [(16 * (zg % 4) + _iota16()) for zg in range(8)]

        def chunk_id(k_):
            return wid + _NW * k_

        def fire_in(k_, b):
            c = chunk_id(k_)
            pltpu.async_copy(
                tT_hbm.at[:, pl.ds(c * _L, _L)], ins[b], isem[b]
            )

        def wait_in(b):
            pltpu.make_async_copy(
                tT_hbm.at[:, pl.ds(0, _L)], ins[b], isem[b]
            ).wait()

        def transpose(b, nq):
            @pl.loop(0, nq, unroll=2)
            def _q_loop(q):
                c0 = jnp.full((16,), 2 * q, jnp.int32)
                c1 = c0 + 1
                for zg in range(8):
                    col = c1 if zg >= 4 else c0
                    v = plsc.load_gather(ins[b], [ridx[zg], col])
                    outs[b][q, pl.ds(16 * zg, 16)] = v * _SCALE

        def fire_out(k_, b):
            c = chunk_id(k_)
            pltpu.async_copy(
                outs[b], tP_hbm.at[pl.ds(c * (_L // 2), _L // 2), :], wsem[b]
            )

        def wait_out(b):
            pltpu.make_async_copy(
                outs[b], tP_hbm.at[pl.ds(0, _L // 2), :], wsem[b]
            ).wait()

        fire_in(0, 0)

        @pl.loop(0, per_w, step=2)
        def _k_loop(k_):
            for p in range(2):
                t = k_ + p
                wait_in(p)

                @pl.when(t + 1 < per_w)
                def _():
                    fire_in(t + 1, 1 - p)

                @pl.when(t >= 2)
                def _():
                    wait_out(p)

                transpose(p, _L // 2)
                fire_out(t, p)

        wait_out(0)
        wait_out(1)

        @pl.when(wid < n_tail)
        def _():
            c = n_full - n_tail + wid
            pltpu.sync_copy(tT_hbm.at[:, pl.ds(c * _L, _L)], ins[0])
            transpose(0, _L // 2)
            pltpu.sync_copy(
                outs[0], tP_hbm.at[pl.ds(c * (_L // 2), _L // 2), :]
            )

        if rem:
            @pl.when(wid == n_tail)
            def _():
                pltpu.sync_copy(tailP_hbm, outs[0].at[pl.ds(0, rem // 2), :])
                pltpu.sync_copy(
                    outs[0].at[pl.ds(0, rem // 2), :],
                    tP_hbm.at[pl.ds(n_full * (_L // 2), rem // 2), :],
                )

    return k(tT, tailP)


def _gather_embed(xT, tP, D):
    S, B = xT.shape
    n_s = S
    sgroups = S // 8

    mesh = plsc.VectorSubcoreMesh(core_axis_name="c", subcore_axis_name="s")

    @functools.partial(
        pl.kernel,
        out_type=jax.ShapeDtypeStruct((S, D, B), jnp.float32),
        mesh=mesh,
        scratch_types=(
            [pltpu.VMEM((S, _L), jnp.int32)]
            + [pltpu.VMEM((_L,), jnp.int32) for _ in range(2)]
            + [pltpu.VMEM((_L,), jnp.int32) for _ in range(2)]
            + [pltpu.VMEM((_L, _L), jnp.float32) for _ in range(2)]
            + [pltpu.VMEM((D, _L), jnp.float32) for _ in range(2)]
            + [pltpu.SemaphoreType.DMA for _ in range(4)]
        ),
        compiler_params=pltpu.CompilerParams(use_tc_tiling_on_sc=True, needs_layout_passes=False),
    )
    def k(xT_hbm, tP_hbm, out_hbm, *refs):
        xall = refs[0]
        gidx = refs[1:3]
        pv = refs[3:5]
        rows = refs[5:7]
        obuf = refs[7:9]
        gsem = refs[9:11]
        wsem = refs[11:13]

        wid = lax.axis_index("s") * 2 + lax.axis_index("c")
        lane0 = wid * _L
        jrow = [(16 * g + _iota16()) for g in range(8)]

        def build_idx(s, b):
            for g in range(8):
                v = xall[s, pl.ds(16 * g, 16)]
                gidx[b][pl.ds(16 * g, 16)] = v >> 1
                pv[b][pl.ds(16 * g, 16)] = (v & 1) << 6

        def fire_gather(b):
            pltpu.async_copy(tP_hbm.at[gidx[b]], rows[b], gsem[b])

        def wait_gather(b):
            pltpu.make_async_copy(tP_hbm.at[gidx[b]], rows[b], gsem[b]).wait()

        def extract(b):
            pvv = [pv[b][pl.ds(16 * g, 16)] for g in range(8)]

            @pl.loop(0, D, unroll=2)
            def _d_loop(d):
                for g in range(8):
                    col = pvv[g] + d
                    obuf[b][d, pl.ds(16 * g, 16)] = plsc.load_gather(
                        rows[b], [jrow[g], col]
                    )

        def fire_write(s, b):
            pltpu.async_copy(
                obuf[b], out_hbm.at[s, :, pl.ds(lane0, _L)], wsem[b]
            )

        def wait_write(b):
            pltpu.make_async_copy(
                obuf[b], out_hbm.at[0, :, pl.ds(lane0, _L)], wsem[b]
            ).wait()

        pltpu.sync_copy(xT_hbm.at[:, pl.ds(lane0, _L)], xall)
        build_idx(0, 0)
        fire_gather(0)

        @pl.loop(0, n_s, step=2)
        def _s_loop(s0):
            for p in range(2):
                s = s0 + p
                wait_gather(p)

                @pl.when(s + 1 < n_s)
                def _():
                    build_idx(s + 1, 1 - p)
                    fire_gather(1 - p)

                @pl.when(s >= 2)
                def _():
                    wait_write(p)

                extract(p)
                fire_write(s, p)

        wait_write(0)
        wait_write(1)

    return k(xT, tP)


def kernel(x, table):
    xT = x.T.astype(jnp.int32)
    tT = table.T
    n_rem = table.shape[0] % _L
    tailP = jnp.reshape(table[-n_rem:] * _SCALE, (n_rem // 2, _L))
    tP = _transpose_table(tT, tailP)
    outT = _gather_embed(xT, tP, table.shape[1])
    return jnp.transpose(outT, (2, 0, 1))


# speedup vs baseline: 1.9727x; 1.9727x over previous
import functools

import jax
import jax.numpy as jnp
from jax import lax
from jax.experimental import pallas as pl
from jax.experimental.pallas import tpu as pltpu
from jax.experimental.pallas import tpu_sc as plsc

_SCALE = 8.0
_NW = 32
_L = 128


def _iota16():
    return lax.iota(jnp.int32, 16)


def _transpose_table(tT, tailP):
    D, V = tT.shape
    n_full = V // _L
    per_w = n_full // _NW
    n_tail = n_full - per_w * _NW
    rem = V - n_full * _L

    mesh = plsc.VectorSubcoreMesh(core_axis_name="c", subcore_axis_name="s")

    @functools.partial(
        pl.kernel,
        out_type=jax.ShapeDtypeStruct((V // 2, _L), jnp.float32),
        mesh=mesh,
        scratch_types=(
            [pltpu.VMEM((D, _L), jnp.float32) for _ in range(2)]
            + [pltpu.VMEM((_L // 2, _L), jnp.float32) for _ in range(2)]
            + [pltpu.SemaphoreType.DMA for _ in range(4)]
        ),
        compiler_params=pltpu.CompilerParams(use_tc_tiling_on_sc=True, needs_layout_passes=False),
    )
    def k(tT_hbm, tailP_hbm, tP_hbm, in0, in1, ot0, ot1, is0, is1, ws0, ws1):
        ins = (in0, in1)
        outs = (ot0, ot1)
        isem = (is0, is1)
        wsem = (ws0, ws1)
        wid = lax.axis_index("s") * 2 + lax.axis_index("c")

        ridx = [(16 * (zg % 4) + _iota16()) for zg in range(8)]

        def chunk_id(k_):
            return wid + _NW * k_

        def fire_in(k_, b):
            c = chunk_id(k_)
            pltpu.async_copy(
                tT_hbm.at[:, pl.ds(c * _L, _L)], ins[b], isem[b]
            )

        def wait_in(b):
            pltpu.make_async_copy(
                tT_hbm.at[:, pl.ds(0, _L)], ins[b], isem[b]
            ).wait()

        def transpose(b, nq):
            @plsc.parallel_loop(0, nq, unroll=4)
            def _q_loop(q):
                c0 = jnp.full((16,), 2 * q, jnp.int32)
                c1 = c0 + 1
                for zg in range(8):
                    col = c1 if zg >= 4 else c0
                    v = plsc.load_gather(ins[b], [ridx[zg], col])
                    outs[b][q, pl.ds(16 * zg, 16)] = v * _SCALE

        def fire_out(k_, b):
            c = chunk_id(k_)
            pltpu.async_copy(
                outs[b], tP_hbm.at[pl.ds(c * (_L // 2), _L // 2), :], wsem[b]
            )

        def wait_out(b):
            pltpu.make_async_copy(
                outs[b], tP_hbm.at[pl.ds(0, _L // 2), :], wsem[b]
            ).wait()

        fire_in(0, 0)

        @pl.loop(0, per_w, step=2)
        def _k_loop(k_):
            for p in range(2):
                t = k_ + p
                wait_in(p)

                @pl.when(t + 1 < per_w)
                def _():
                    fire_in(t + 1, 1 - p)

                @pl.when(t >= 2)
                def _():
                    wait_out(p)

                transpose(p, _L // 2)
                fire_out(t, p)

        wait_out(0)
        wait_out(1)

        @pl.when(wid < n_tail)
        def _():
            c = n_full - n_tail + wid
            pltpu.sync_copy(tT_hbm.at[:, pl.ds(c * _L, _L)], ins[0])
            transpose(0, _L // 2)
            pltpu.sync_copy(
                outs[0], tP_hbm.at[pl.ds(c * (_L // 2), _L // 2), :]
            )

        if rem:
            @pl.when(wid == n_tail)
            def _():
                pltpu.sync_copy(tailP_hbm, outs[0].at[pl.ds(0, rem // 2), :])
                pltpu.sync_copy(
                    outs[0].at[pl.ds(0, rem // 2), :],
                    tP_hbm.at[pl.ds(n_full * (_L // 2), rem // 2), :],
                )

    return k(tT, tailP)


def _gather_embed(xT, tP, D):
    S, B = xT.shape
    n_s = S
    sgroups = S // 8

    mesh = plsc.VectorSubcoreMesh(core_axis_name="c", subcore_axis_name="s")

    @functools.partial(
        pl.kernel,
        out_type=jax.ShapeDtypeStruct((S, D, B), jnp.float32),
        mesh=mesh,
        scratch_types=(
            [pltpu.VMEM((S, _L), jnp.int32)]
            + [pltpu.VMEM((_L,), jnp.int32) for _ in range(2)]
            + [pltpu.VMEM((_L,), jnp.int32) for _ in range(2)]
            + [pltpu.VMEM((_L, _L), jnp.float32) for _ in range(2)]
            + [pltpu.VMEM((D, _L), jnp.float32) for _ in range(2)]
            + [pltpu.SemaphoreType.DMA for _ in range(4)]
        ),
        compiler_params=pltpu.CompilerParams(use_tc_tiling_on_sc=True, needs_layout_passes=False),
    )
    def k(xT_hbm, tP_hbm, out_hbm, *refs):
        xall = refs[0]
        gidx = refs[1:3]
        pv = refs[3:5]
        rows = refs[5:7]
        obuf = refs[7:9]
        gsem = refs[9:11]
        wsem = refs[11:13]

        wid = lax.axis_index("s") * 2 + lax.axis_index("c")
        lane0 = wid * _L
        jrow = [(16 * g + _iota16()) for g in range(8)]

        def build_idx(s, b):
            for g in range(8):
                v = xall[s, pl.ds(16 * g, 16)]
                gidx[b][pl.ds(16 * g, 16)] = v >> 1
                pv[b][pl.ds(16 * g, 16)] = (v & 1) << 6

        def fire_gather(b):
            pltpu.async_copy(tP_hbm.at[gidx[b]], rows[b], gsem[b])

        def wait_gather(b):
            pltpu.make_async_copy(tP_hbm.at[gidx[b]], rows[b], gsem[b]).wait()

        def extract(b):
            pvv = [pv[b][pl.ds(16 * g, 16)] for g in range(8)]

            @plsc.parallel_loop(0, D, unroll=4)
            def _d_loop(d):
                for g in range(8):
                    col = pvv[g] + d
                    obuf[b][d, pl.ds(16 * g, 16)] = plsc.load_gather(
                        rows[b], [jrow[g], col]
                    )

        def fire_write(s, b):
            pltpu.async_copy(
                obuf[b], out_hbm.at[s, :, pl.ds(lane0, _L)], wsem[b]
            )

        def wait_write(b):
            pltpu.make_async_copy(
                obuf[b], out_hbm.at[0, :, pl.ds(lane0, _L)], wsem[b]
            ).wait()

        pltpu.sync_copy(xT_hbm.at[:, pl.ds(lane0, _L)], xall)
        build_idx(0, 0)
        fire_gather(0)

        @pl.loop(0, n_s, step=2)
        def _s_loop(s0):
            for p in range(2):
                s = s0 + p
                wait_gather(p)

                @pl.when(s + 1 < n_s)
                def _():
                    build_idx(s + 1, 1 - p)
                    fire_gather(1 - p)

                @pl.when(s >= 2)
                def _():
                    wait_write(p)

                extract(p)
                fire_write(s, p)

        wait_write(0)
        wait_write(1)

    return k(xT, tP)


def kernel(x, table):
    xT = x.T.astype(jnp.int32)
    tT = table.T
    n_rem = table.shape[0] % _L
    tailP = jnp.reshape(table[-n_rem:] * _SCALE, (n_rem // 2, _L))
    tP = _transpose_table(tT, tailP)
    outT = _gather_embed(xT, tP, table.shape[1])
    return jnp.transpose(outT, (2, 0, 1))


# speedup vs baseline: 1.9774x; 1.0024x over previous
import functools

import jax
import jax.numpy as jnp
from jax import lax
from jax.experimental import pallas as pl
from jax.experimental.pallas import tpu as pltpu
from jax.experimental.pallas import tpu_sc as plsc

_SCALE = 8.0
_NW = 32
_L = 128


def _iota16():
    return lax.iota(jnp.int32, 16)


def _transpose_table(tT, tailP):
    D, V = tT.shape
    n_full = V // _L
    per_w = n_full // _NW
    n_tail = n_full - per_w * _NW
    rem = V - n_full * _L

    mesh = plsc.VectorSubcoreMesh(core_axis_name="c", subcore_axis_name="s")

    @functools.partial(
        pl.kernel,
        out_type=jax.ShapeDtypeStruct((V // 2, _L), jnp.float32),
        mesh=mesh,
        scratch_types=(
            [pltpu.VMEM((D, _L), jnp.float32) for _ in range(2)]
            + [pltpu.VMEM((_L // 2, _L), jnp.float32) for _ in range(2)]
            + [pltpu.SemaphoreType.DMA for _ in range(4)]
        ),
        compiler_params=pltpu.CompilerParams(use_tc_tiling_on_sc=True, needs_layout_passes=False),
    )
    def k(tT_hbm, tailP_hbm, tP_hbm, in0, in1, ot0, ot1, is0, is1, ws0, ws1):
        ins = (in0, in1)
        outs = (ot0, ot1)
        isem = (is0, is1)
        wsem = (ws0, ws1)
        wid = lax.axis_index("s") * 2 + lax.axis_index("c")

        ridx = [(16 * (zg % 4) + _iota16()) for zg in range(8)]

        def chunk_id(k_):
            return wid + _NW * k_

        def fire_in(k_, b):
            c = chunk_id(k_)
            pltpu.async_copy(
                tT_hbm.at[:, pl.ds(c * _L, _L)], ins[b], isem[b]
            )

        def wait_in(b):
            pltpu.make_async_copy(
                tT_hbm.at[:, pl.ds(0, _L)], ins[b], isem[b]
            ).wait()

        def transpose(b, nq):
            @plsc.parallel_loop(0, nq, unroll=8)
            def _q_loop(q):
                c0 = jnp.full((16,), 2 * q, jnp.int32)
                c1 = c0 + 1
                for zg in range(8):
                    col = c1 if zg >= 4 else c0
                    v = plsc.load_gather(ins[b], [ridx[zg], col])
                    outs[b][q, pl.ds(16 * zg, 16)] = v * _SCALE

        def fire_out(k_, b):
            c = chunk_id(k_)
            pltpu.async_copy(
                outs[b], tP_hbm.at[pl.ds(c * (_L // 2), _L // 2), :], wsem[b]
            )

        def wait_out(b):
            pltpu.make_async_copy(
                outs[b], tP_hbm.at[pl.ds(0, _L // 2), :], wsem[b]
            ).wait()

        fire_in(0, 0)

        @pl.loop(0, per_w, step=2)
        def _k_loop(k_):
            for p in range(2):
                t = k_ + p
                wait_in(p)

                @pl.when(t + 1 < per_w)
                def _():
                    fire_in(t + 1, 1 - p)

                @pl.when(t >= 2)
                def _():
                    wait_out(p)

                transpose(p, _L // 2)
                fire_out(t, p)

        wait_out(0)
        wait_out(1)

        @pl.when(wid < n_tail)
        def _():
            c = n_full - n_tail + wid
            pltpu.sync_copy(tT_hbm.at[:, pl.ds(c * _L, _L)], ins[0])
            transpose(0, _L // 2)
            pltpu.sync_copy(
                outs[0], tP_hbm.at[pl.ds(c * (_L // 2), _L // 2), :]
            )

        if rem:
            @pl.when(wid == n_tail)
            def _():
                pltpu.sync_copy(tailP_hbm, outs[0].at[pl.ds(0, rem // 2), :])
                pltpu.sync_copy(
                    outs[0].at[pl.ds(0, rem // 2), :],
                    tP_hbm.at[pl.ds(n_full * (_L // 2), rem // 2), :],
                )

    return k(tT, tailP)


def _gather_embed(xT, tP, D):
    S, B = xT.shape
    n_s = S
    sgroups = S // 8

    mesh = plsc.VectorSubcoreMesh(core_axis_name="c", subcore_axis_name="s")

    @functools.partial(
        pl.kernel,
        out_type=jax.ShapeDtypeStruct((S, D, B), jnp.float32),
        mesh=mesh,
        scratch_types=(
            [pltpu.VMEM((S, _L), jnp.int32)]
            + [pltpu.VMEM((_L,), jnp.int32) for _ in range(2)]
            + [pltpu.VMEM((_L,), jnp.int32) for _ in range(2)]
            + [pltpu.VMEM((_L, _L), jnp.float32) for _ in range(2)]
            + [pltpu.VMEM((D, _L), jnp.float32) for _ in range(2)]
            + [pltpu.SemaphoreType.DMA for _ in range(4)]
        ),
        compiler_params=pltpu.CompilerParams(use_tc_tiling_on_sc=True, needs_layout_passes=False),
    )
    def k(xT_hbm, tP_hbm, out_hbm, *refs):
        xall = refs[0]
        gidx = refs[1:3]
        pv = refs[3:5]
        rows = refs[5:7]
        obuf = refs[7:9]
        gsem = refs[9:11]
        wsem = refs[11:13]

        wid = lax.axis_index("s") * 2 + lax.axis_index("c")
        lane0 = wid * _L
        jrow = [(16 * g + _iota16()) for g in range(8)]

        def build_idx(s, b):
            for g in range(8):
                v = xall[s, pl.ds(16 * g, 16)]
                gidx[b][pl.ds(16 * g, 16)] = v >> 1
                pv[b][pl.ds(16 * g, 16)] = (v & 1) << 6

        def fire_gather(b):
            pltpu.async_copy(tP_hbm.at[gidx[b]], rows[b], gsem[b])

        def wait_gather(b):
            pltpu.make_async_copy(tP_hbm.at[gidx[b]], rows[b], gsem[b]).wait()

        def extract(b):
            pvv = [pv[b][pl.ds(16 * g, 16)] for g in range(8)]

            @plsc.parallel_loop(0, D, unroll=8)
            def _d_loop(d):
                for g in range(8):
                    col = pvv[g] + d
                    obuf[b][d, pl.ds(16 * g, 16)] = plsc.load_gather(
                        rows[b], [jrow[g], col]
                    )

        def fire_write(s, b):
            pltpu.async_copy(
                obuf[b], out_hbm.at[s, :, pl.ds(lane0, _L)], wsem[b]
            )

        def wait_write(b):
            pltpu.make_async_copy(
                obuf[b], out_hbm.at[0, :, pl.ds(lane0, _L)], wsem[b]
            ).wait()

        pltpu.sync_copy(xT_hbm.at[:, pl.ds(lane0, _L)], xall)
        build_idx(0, 0)
        fire_gather(0)

        @pl.loop(0, n_s, step=2)
        def _s_loop(s0):
            for p in range(2):
                s = s0 + p
                wait_gather(p)

                @pl.when(s + 1 < n_s)
                def _():
                    build_idx(s + 1, 1 - p)
                    fire_gather(1 - p)

                @pl.when(s >= 2)
                def _():
                    wait_write(p)

                extract(p)
                fire_write(s, p)

        wait_write(0)
        wait_write(1)

    return k(xT, tP)


def kernel(x, table):
    xT = x.T.astype(jnp.int32)
    tT = table.T
    n_rem = table.shape[0] % _L
    tailP = jnp.reshape(table[-n_rem:] * _SCALE, (n_rem // 2, _L))
    tP = _transpose_table(tT, tailP)
    outT = _gather_embed(xT, tP, table.shape[1])
    return jnp.transpose(outT, (2, 0, 1))


# speedup vs baseline: 4.2014x; 2.1247x over previous
import functools

import jax
import jax.numpy as jnp
from jax import lax
from jax.experimental import pallas as pl
from jax.experimental.pallas import tpu as pltpu
from jax.experimental.pallas import tpu_sc as plsc

_SCALE = 8.0
_NW = 32
_L = 128


def _iota16():
    return lax.iota(jnp.int32, 16)


def _transpose_table(tT, tailP):
    D, V = tT.shape
    n_full = V // _L
    per_w = n_full // _NW
    n_tail = n_full - per_w * _NW
    rem = V - n_full * _L

    mesh = plsc.VectorSubcoreMesh(core_axis_name="c", subcore_axis_name="s")

    @functools.partial(
        pl.kernel,
        out_type=jax.ShapeDtypeStruct((V // 2, _L), jnp.float32),
        mesh=mesh,
        scratch_types=(
            [pltpu.VMEM((D, _L), jnp.float32) for _ in range(2)]
            + [pltpu.VMEM((_L // 2, _L), jnp.float32) for _ in range(2)]
            + [pltpu.SemaphoreType.DMA for _ in range(4)]
        ),
        compiler_params=pltpu.CompilerParams(use_tc_tiling_on_sc=True, needs_layout_passes=False),
    )
    def k(tT_hbm, tailP_hbm, tP_hbm, in0, in1, ot0, ot1, is0, is1, ws0, ws1):
        ins = (in0, in1)
        outs = (ot0, ot1)
        isem = (is0, is1)
        wsem = (ws0, ws1)
        wid = lax.axis_index("s") * 2 + lax.axis_index("c")

        ridx = [(16 * (zg % 4) + _iota16()) for zg in range(8)]

        def chunk_id(k_):
            return wid + _NW * k_

        def fire_in(k_, b):
            c = chunk_id(k_)
            pltpu.async_copy(
                tT_hbm.at[:, pl.ds(c * _L, _L)], ins[b], isem[b]
            )

        def wait_in(b):
            pltpu.make_async_copy(
                tT_hbm.at[:, pl.ds(0, _L)], ins[b], isem[b]
            ).wait()

        def transpose(b, nq):
            nC = nq // 8
            iota = _iota16()
            par64 = (iota & 1) << 6
            qv = [8 * C + (iota >> 1) for C in range(nC)]
            cv = [16 * C + iota for C in range(nC)]

            @plsc.parallel_loop(0, 16, unroll=2)
            def _rot(rot):
                rotv = (rot + iota) & 15
                for R in range(4):
                    rv = 16 * R + rotv
                    zv = rv + par64
                    for C in range(nC):
                        val = plsc.load_gather(ins[b], [rv, cv[C]])
                        plsc.store_scatter(
                            outs[b], [qv[C], zv], val * _SCALE
                        )

        def fire_out(k_, b):
            c = chunk_id(k_)
            pltpu.async_copy(
                outs[b], tP_hbm.at[pl.ds(c * (_L // 2), _L // 2), :], wsem[b]
            )

        def wait_out(b):
            pltpu.make_async_copy(
                outs[b], tP_hbm.at[pl.ds(0, _L // 2), :], wsem[b]
            ).wait()

        fire_in(0, 0)

        @pl.loop(0, per_w, step=2)
        def _k_loop(k_):
            for p in range(2):
                t = k_ + p
                wait_in(p)

                @pl.when(t + 1 < per_w)
                def _():
                    fire_in(t + 1, 1 - p)

                @pl.when(t >= 2)
                def _():
                    wait_out(p)

                transpose(p, _L // 2)
                fire_out(t, p)

        wait_out(0)
        wait_out(1)

        @pl.when(wid < n_tail)
        def _():
            c = n_full - n_tail + wid
            pltpu.sync_copy(tT_hbm.at[:, pl.ds(c * _L, _L)], ins[0])
            transpose(0, _L // 2)
            pltpu.sync_copy(
                outs[0], tP_hbm.at[pl.ds(c * (_L // 2), _L // 2), :]
            )

        if rem:
            @pl.when(wid == n_tail)
            def _():
                pltpu.sync_copy(tailP_hbm, outs[0].at[pl.ds(0, rem // 2), :])
                pltpu.sync_copy(
                    outs[0].at[pl.ds(0, rem // 2), :],
                    tP_hbm.at[pl.ds(n_full * (_L // 2), rem // 2), :],
                )

    return k(tT, tailP)


def _gather_embed(xT, tP, D):
    S, B = xT.shape
    n_s = S
    sgroups = S // 8

    mesh = plsc.VectorSubcoreMesh(core_axis_name="c", subcore_axis_name="s")

    @functools.partial(
        pl.kernel,
        out_type=jax.ShapeDtypeStruct((S, D, B), jnp.float32),
        mesh=mesh,
        scratch_types=(
            [pltpu.VMEM((S, _L), jnp.int32)]
            + [pltpu.VMEM((_L,), jnp.int32) for _ in range(2)]
            + [pltpu.VMEM((_L,), jnp.int32) for _ in range(2)]
            + [pltpu.VMEM((_L, _L), jnp.float32) for _ in range(2)]
            + [pltpu.VMEM((D, _L), jnp.float32) for _ in range(2)]
            + [pltpu.SemaphoreType.DMA for _ in range(4)]
        ),
        compiler_params=pltpu.CompilerParams(use_tc_tiling_on_sc=True, needs_layout_passes=False),
    )
    def k(xT_hbm, tP_hbm, out_hbm, *refs):
        xall = refs[0]
        gidx = refs[1:3]
        pv = refs[3:5]
        rows = refs[5:7]
        obuf = refs[7:9]
        gsem = refs[9:11]
        wsem = refs[11:13]

        wid = lax.axis_index("s") * 2 + lax.axis_index("c")
        lane0 = wid * _L
        jrow = [(16 * g + _iota16()) for g in range(8)]

        def build_idx(s, b):
            for g in range(8):
                v = xall[s, pl.ds(16 * g, 16)]
                gidx[b][pl.ds(16 * g, 16)] = v >> 1
                pv[b][pl.ds(16 * g, 16)] = (v & 1) << 6

        def fire_gather(b):
            pltpu.async_copy(tP_hbm.at[gidx[b]], rows[b], gsem[b])

        def wait_gather(b):
            pltpu.make_async_copy(tP_hbm.at[gidx[b]], rows[b], gsem[b]).wait()

        def extract(b):
            iota = _iota16()
            pvv = [pv[b][pl.ds(16 * g, 16)] for g in range(8)]

            @plsc.parallel_loop(0, 16, unroll=2)
            def _rot(rot):
                rotv = (rot + iota) & 15
                for g in range(8):
                    t = rotv + pvv[g]
                    for K in range(4):
                        dv = 16 * K + rotv
                        val = plsc.load_gather(rows[b], [jrow[g], t + 16 * K])
                        plsc.store_scatter(obuf[b], [dv, jrow[g]], val)

        def fire_write(s, b):
            pltpu.async_copy(
                obuf[b], out_hbm.at[s, :, pl.ds(lane0, _L)], wsem[b]
            )

        def wait_write(b):
            pltpu.make_async_copy(
                obuf[b], out_hbm.at[0, :, pl.ds(lane0, _L)], wsem[b]
            ).wait()

        pltpu.sync_copy(xT_hbm.at[:, pl.ds(lane0, _L)], xall)
        build_idx(0, 0)
        fire_gather(0)

        @pl.loop(0, n_s, step=2)
        def _s_loop(s0):
            for p in range(2):
                s = s0 + p
                wait_gather(p)

                @pl.when(s + 1 < n_s)
                def _():
                    build_idx(s + 1, 1 - p)
                    fire_gather(1 - p)

                @pl.when(s >= 2)
                def _():
                    wait_write(p)

                extract(p)
                fire_write(s, p)

        wait_write(0)
        wait_write(1)

    return k(xT, tP)


def kernel(x, table):
    xT = x.T.astype(jnp.int32)
    tT = table.T
    n_rem = table.shape[0] % _L
    tailP = jnp.reshape(table[-n_rem:] * _SCALE, (n_rem // 2, _L))
    tP = _transpose_table(tT, tailP)
    outT = _gather_embed(xT, tP, table.shape[1])
    return jnp.transpose(outT, (2, 0, 1))


# speedup vs baseline: 4.2374x; 1.0086x over previous
import functools

import jax
import jax.numpy as jnp
from jax import lax
from jax.experimental import pallas as pl
from jax.experimental.pallas import tpu as pltpu
from jax.experimental.pallas import tpu_sc as plsc

_SCALE = 8.0
_NW = 32
_L = 128


def _iota16():
    return lax.iota(jnp.int32, 16)


def _transpose_table(tT, tailP):
    D, V = tT.shape
    n_full = V // _L
    per_w = n_full // _NW
    n_tail = n_full - per_w * _NW
    rem = V - n_full * _L

    mesh = plsc.VectorSubcoreMesh(core_axis_name="c", subcore_axis_name="s")

    @functools.partial(
        pl.kernel,
        out_type=jax.ShapeDtypeStruct((V // 2, _L), jnp.float32),
        mesh=mesh,
        scratch_types=(
            [pltpu.VMEM((D, _L), jnp.float32) for _ in range(2)]
            + [pltpu.VMEM((_L // 2, _L), jnp.float32) for _ in range(2)]
            + [pltpu.SemaphoreType.DMA for _ in range(4)]
        ),
        compiler_params=pltpu.CompilerParams(use_tc_tiling_on_sc=True, needs_layout_passes=False),
    )
    def k(tT_hbm, tailP_hbm, tP_hbm, in0, in1, ot0, ot1, is0, is1, ws0, ws1):
        ins = (in0, in1)
        outs = (ot0, ot1)
        isem = (is0, is1)
        wsem = (ws0, ws1)
        wid = lax.axis_index("s") * 2 + lax.axis_index("c")

        ridx = [(16 * (zg % 4) + _iota16()) for zg in range(8)]

        def chunk_id(k_):
            return wid + _NW * k_

        def fire_in(k_, b):
            c = chunk_id(k_)
            pltpu.async_copy(
                tT_hbm.at[:, pl.ds(c * _L, _L)], ins[b], isem[b]
            )

        def wait_in(b):
            pltpu.make_async_copy(
                tT_hbm.at[:, pl.ds(0, _L)], ins[b], isem[b]
            ).wait()

        def transpose(b, nq):
            nC = nq // 8
            iota = _iota16()
            par64 = (iota & 1) << 6
            qv = [8 * C + (iota >> 1) for C in range(nC)]
            cv = [16 * C + iota for C in range(nC)]

            @plsc.parallel_loop(0, 16, unroll=4)
            def _rot(rot):
                rotv = (rot + iota) & 15
                for R in range(4):
                    rv = 16 * R + rotv
                    zv = rv + par64
                    for C in range(nC):
                        val = plsc.load_gather(ins[b], [rv, cv[C]])
                        plsc.store_scatter(
                            outs[b], [qv[C], zv], val * _SCALE
                        )

        def fire_out(k_, b):
            c = chunk_id(k_)
            pltpu.async_copy(
                outs[b], tP_hbm.at[pl.ds(c * (_L // 2), _L // 2), :], wsem[b]
            )

        def wait_out(b):
            pltpu.make_async_copy(
                outs[b], tP_hbm.at[pl.ds(0, _L // 2), :], wsem[b]
            ).wait()

        fire_in(0, 0)

        @pl.loop(0, per_w, step=2)
        def _k_loop(k_):
            for p in range(2):
                t = k_ + p
                wait_in(p)

                @pl.when(t + 1 < per_w)
                def _():
                    fire_in(t + 1, 1 - p)

                @pl.when(t >= 2)
                def _():
                    wait_out(p)

                transpose(p, _L // 2)
                fire_out(t, p)

        wait_out(0)
        wait_out(1)

        @pl.when(wid < n_tail)
        def _():
            c = n_full - n_tail + wid
            pltpu.sync_copy(tT_hbm.at[:, pl.ds(c * _L, _L)], ins[0])
            transpose(0, _L // 2)
            pltpu.sync_copy(
                outs[0], tP_hbm.at[pl.ds(c * (_L // 2), _L // 2), :]
            )

        if rem:
            @pl.when(wid == n_tail)
            def _():
                pltpu.sync_copy(tailP_hbm, outs[0].at[pl.ds(0, rem // 2), :])
                pltpu.sync_copy(
                    outs[0].at[pl.ds(0, rem // 2), :],
                    tP_hbm.at[pl.ds(n_full * (_L // 2), rem // 2), :],
                )

    return k(tT, tailP)


def _gather_embed(xT, tP, D):
    S, B = xT.shape
    n_s = S
    sgroups = S // 8

    mesh = plsc.VectorSubcoreMesh(core_axis_name="c", subcore_axis_name="s")

    @functools.partial(
        pl.kernel,
        out_type=jax.ShapeDtypeStruct((S, D, B), jnp.float32),
        mesh=mesh,
        scratch_types=(
            [pltpu.VMEM((S, _L), jnp.int32)]
            + [pltpu.VMEM((_L,), jnp.int32) for _ in range(2)]
            + [pltpu.VMEM((_L,), jnp.int32) for _ in range(2)]
            + [pltpu.VMEM((_L, _L), jnp.float32) for _ in range(2)]
            + [pltpu.VMEM((D, _L), jnp.float32) for _ in range(2)]
            + [pltpu.SemaphoreType.DMA for _ in range(4)]
        ),
        compiler_params=pltpu.CompilerParams(use_tc_tiling_on_sc=True, needs_layout_passes=False),
    )
    def k(xT_hbm, tP_hbm, out_hbm, *refs):
        xall = refs[0]
        gidx = refs[1:3]
        pv = refs[3:5]
        rows = refs[5:7]
        obuf = refs[7:9]
        gsem = refs[9:11]
        wsem = refs[11:13]

        wid = lax.axis_index("s") * 2 + lax.axis_index("c")
        lane0 = wid * _L
        jrow = [(16 * g + _iota16()) for g in range(8)]

        def build_idx(s, b):
            for g in range(8):
                v = xall[s, pl.ds(16 * g, 16)]
                gidx[b][pl.ds(16 * g, 16)] = v >> 1
                pv[b][pl.ds(16 * g, 16)] = (v & 1) << 6

        def fire_gather(b):
            pltpu.async_copy(tP_hbm.at[gidx[b]], rows[b], gsem[b])

        def wait_gather(b):
            pltpu.make_async_copy(tP_hbm.at[gidx[b]], rows[b], gsem[b]).wait()

        def extract(b):
            iota = _iota16()
            pvv = [pv[b][pl.ds(16 * g, 16)] for g in range(8)]

            @plsc.parallel_loop(0, 16, unroll=4)
            def _rot(rot):
                rotv = (rot + iota) & 15
                for g in range(8):
                    t = rotv + pvv[g]
                    for K in range(4):
                        dv = 16 * K + rotv
                        val = plsc.load_gather(rows[b], [jrow[g], t + 16 * K])
                        plsc.store_scatter(obuf[b], [dv, jrow[g]], val)

        def fire_write(s, b):
            pltpu.async_copy(
                obuf[b], out_hbm.at[s, :, pl.ds(lane0, _L)], wsem[b]
            )

        def wait_write(b):
            pltpu.make_async_copy(
                obuf[b], out_hbm.at[0, :, pl.ds(lane0, _L)], wsem[b]
            ).wait()

        pltpu.sync_copy(xT_hbm.at[:, pl.ds(lane0, _L)], xall)
        build_idx(0, 0)
        fire_gather(0)

        @pl.loop(0, n_s, step=2)
        def _s_loop(s0):
            for p in range(2):
                s = s0 + p
                wait_gather(p)

                @pl.when(s + 1 < n_s)
                def _():
                    build_idx(s + 1, 1 - p)
                    fire_gather(1 - p)

                @pl.when(s >= 2)
                def _():
                    wait_write(p)

                extract(p)
                fire_write(s, p)

        wait_write(0)
        wait_write(1)

    return k(xT, tP)


def kernel(x, table):
    xT = x.T.astype(jnp.int32)
    tT = table.T
    n_rem = table.shape[0] % _L
    tailP = jnp.reshape(table[-n_rem:] * _SCALE, (n_rem // 2, _L))
    tP = _transpose_table(tT, tailP)
    outT = _gather_embed(xT, tP, table.shape[1])
    return jnp.transpose(outT, (2, 0, 1))


# speedup vs baseline: 4.7214x; 1.1142x over previous
import functools

import jax
import jax.numpy as jnp
from jax import lax
from jax.experimental import pallas as pl
from jax.experimental.pallas import tpu as pltpu
from jax.experimental.pallas import tpu_sc as plsc

_SCALE = 8.0
_NW = 32
_L = 128


def _iota16():
    return lax.iota(jnp.int32, 16)


def _transpose_table(tT, tailP):
    D, V = tT.shape
    n_full = V // _L
    per_w = n_full // _NW
    n_tail = n_full - per_w * _NW
    rem = V - n_full * _L

    mesh = plsc.VectorSubcoreMesh(core_axis_name="c", subcore_axis_name="s")

    @functools.partial(
        pl.kernel,
        out_type=jax.ShapeDtypeStruct((V // 2, _L), jnp.float32),
        mesh=mesh,
        scratch_types=(
            [pltpu.VMEM((D, _L), jnp.float32) for _ in range(2)]
            + [pltpu.VMEM((_L // 2, _L), jnp.float32) for _ in range(2)]
            + [pltpu.SemaphoreType.DMA for _ in range(4)]
        ),
        compiler_params=pltpu.CompilerParams(use_tc_tiling_on_sc=True, needs_layout_passes=False),
    )
    def k(tT_hbm, tailP_hbm, tP_hbm, in0, in1, ot0, ot1, is0, is1, ws0, ws1):
        ins = (in0, in1)
        outs = (ot0, ot1)
        isem = (is0, is1)
        wsem = (ws0, ws1)
        wid = lax.axis_index("s") * 2 + lax.axis_index("c")

        ridx = [(16 * (zg % 4) + _iota16()) for zg in range(8)]

        def chunk_id(k_):
            return wid + _NW * k_

        def fire_in(k_, b):
            c = chunk_id(k_)
            pltpu.async_copy(
                tT_hbm.at[:, pl.ds(c * _L, _L)], ins[b], isem[b]
            )

        def wait_in(b):
            pltpu.make_async_copy(
                tT_hbm.at[:, pl.ds(0, _L)], ins[b], isem[b]
            ).wait()

        def transpose(b, nq):
            nC = nq // 8
            iota = _iota16()
            par64 = (iota & 1) << 6
            qv = [8 * C + (iota >> 1) for C in range(nC)]
            cv = [16 * C + iota for C in range(nC)]

            @plsc.parallel_loop(0, 16, unroll=4)
            def _rot(rot):
                rotv = (rot + iota) & 15
                for R in range(4):
                    rv = 16 * R + rotv
                    zv = rv + par64
                    for C in range(nC):
                        val = plsc.load_gather(ins[b], [rv, cv[C]])
                        plsc.store_scatter(
                            outs[b], [qv[C], zv], val * _SCALE
                        )

        def fire_out(k_, b):
            c = chunk_id(k_)
            pltpu.async_copy(
                outs[b], tP_hbm.at[pl.ds(c * (_L // 2), _L // 2), :], wsem[b]
            )

        def wait_out(b):
            pltpu.make_async_copy(
                outs[b], tP_hbm.at[pl.ds(0, _L // 2), :], wsem[b]
            ).wait()

        fire_in(0, 0)

        @pl.loop(0, per_w, step=2)
        def _k_loop(k_):
            for p in range(2):
                t = k_ + p

                @pl.when(t + 1 < per_w)
                def _():
                    fire_in(t + 1, 1 - p)

                wait_in(p)

                @pl.when(t >= 2)
                def _():
                    wait_out(p)

                transpose(p, _L // 2)
                fire_out(t, p)

        wait_out(0)
        wait_out(1)

        @pl.when(wid < n_tail)
        def _():
            c = n_full - n_tail + wid
            pltpu.sync_copy(tT_hbm.at[:, pl.ds(c * _L, _L)], ins[0])
            transpose(0, _L // 2)
            pltpu.sync_copy(
                outs[0], tP_hbm.at[pl.ds(c * (_L // 2), _L // 2), :]
            )

        if rem:
            @pl.when(wid == n_tail)
            def _():
                pltpu.sync_copy(tailP_hbm, outs[0].at[pl.ds(0, rem // 2), :])
                pltpu.sync_copy(
                    outs[0].at[pl.ds(0, rem // 2), :],
                    tP_hbm.at[pl.ds(n_full * (_L // 2), rem // 2), :],
                )

    return k(tT, tailP)


def _gather_embed(xT, tP, D):
    S, B = xT.shape
    n_s = S
    sgroups = S // 8

    mesh = plsc.VectorSubcoreMesh(core_axis_name="c", subcore_axis_name="s")

    @functools.partial(
        pl.kernel,
        out_type=jax.ShapeDtypeStruct((S, D, B), jnp.float32),
        mesh=mesh,
        scratch_types=(
            [pltpu.VMEM((S, _L), jnp.int32)]
            + [pltpu.VMEM((_L,), jnp.int32) for _ in range(2)]
            + [pltpu.VMEM((_L,), jnp.int32) for _ in range(2)]
            + [pltpu.VMEM((_L, _L), jnp.float32) for _ in range(2)]
            + [pltpu.VMEM((D, _L), jnp.float32) for _ in range(2)]
            + [pltpu.SemaphoreType.DMA for _ in range(4)]
        ),
        compiler_params=pltpu.CompilerParams(use_tc_tiling_on_sc=True, needs_layout_passes=False),
    )
    def k(xT_hbm, tP_hbm, out_hbm, *refs):
        xall = refs[0]
        gidx = refs[1:3]
        pv = refs[3:5]
        rows = refs[5:7]
        obuf = refs[7:9]
        gsem = refs[9:11]
        wsem = refs[11:13]

        wid = lax.axis_index("s") * 2 + lax.axis_index("c")
        lane0 = wid * _L
        jrow = [(16 * g + _iota16()) for g in range(8)]

        def build_idx(s, b):
            for g in range(8):
                v = xall[s, pl.ds(16 * g, 16)]
                gidx[b][pl.ds(16 * g, 16)] = v >> 1
                pv[b][pl.ds(16 * g, 16)] = (v & 1) << 6

        def fire_gather(b):
            pltpu.async_copy(tP_hbm.at[gidx[b]], rows[b], gsem[b])

        def wait_gather(b):
            pltpu.make_async_copy(tP_hbm.at[gidx[b]], rows[b], gsem[b]).wait()

        def extract(b):
            iota = _iota16()
            pvv = [pv[b][pl.ds(16 * g, 16)] for g in range(8)]

            @plsc.parallel_loop(0, 16, unroll=4)
            def _rot(rot):
                rotv = (rot + iota) & 15
                for g in range(8):
                    t = rotv + pvv[g]
                    for K in range(4):
                        dv = 16 * K + rotv
                        val = plsc.load_gather(rows[b], [jrow[g], t + 16 * K])
                        plsc.store_scatter(obuf[b], [dv, jrow[g]], val)

        def fire_write(s, b):
            pltpu.async_copy(
                obuf[b], out_hbm.at[s, :, pl.ds(lane0, _L)], wsem[b]
            )

        def wait_write(b):
            pltpu.make_async_copy(
                obuf[b], out_hbm.at[0, :, pl.ds(lane0, _L)], wsem[b]
            ).wait()

        pltpu.sync_copy(xT_hbm.at[:, pl.ds(lane0, _L)], xall)
        build_idx(0, 0)
        fire_gather(0)

        @pl.loop(0, n_s, step=2)
        def _s_loop(s0):
            for p in range(2):
                s = s0 + p

                @pl.when(s + 1 < n_s)
                def _():
                    build_idx(s + 1, 1 - p)
                    fire_gather(1 - p)

                wait_gather(p)

                @pl.when(s >= 2)
                def _():
                    wait_write(p)

                extract(p)
                fire_write(s, p)

        wait_write(0)
        wait_write(1)

    return k(xT, tP)


def kernel(x, table):
    xT = x.T.astype(jnp.int32)
    tT = table.T
    n_rem = table.shape[0] % _L
    tailP = jnp.reshape(table[-n_rem:] * _SCALE, (n_rem // 2, _L))
    tP = _transpose_table(tT, tailP)
    outT = _gather_embed(xT, tP, table.shape[1])
    return jnp.transpose(outT, (2, 0, 1))
